# baseline (speedup 1.0000x reference)
import jax
import jax.numpy as jnp
from jax.experimental import pallas as pl
from jax.experimental.pallas import tpu as pltpu


def _gconv_kernel(emb_ref, wt_ref, bfc_ref, meta_a_ref, meta_b_ref,
                  bias_ref, a_ref, out_ref, feat_ref):
    @pl.when(pl.program_id(0) == 0)
    def _():
        feat_ref[...] = jnp.dot(
            emb_ref[...].astype(jnp.bfloat16),
            wt_ref[...].astype(jnp.bfloat16),
            preferred_element_type=jnp.float32,
        ) + bfc_ref[...]

    bm = meta_a_ref.shape[0]
    for half, ref in ((0, meta_a_ref), (1, meta_b_ref)):
        acc = jax.lax.dot_general(
            ref[...],
            feat_ref[...],
            (((1,), (0,)), ((), ())),
            precision=jax.lax.Precision.DEFAULT,
            preferred_element_type=jnp.float32,
        )
        r = acc + bias_ref[...]
        out_ref[pl.ds(half * bm, bm), :] = jnp.where(
            r >= 0, r, a_ref[0, 0] * r)


def kernel(emb, meta, W, b_fc, bias, prelu_a):
    n, in_ch = emb.shape
    out_ch = W.shape[0]

    bm = 200
    grid = (n // (2 * bm),)
    out = pl.pallas_call(
        _gconv_kernel,
        grid=grid,
        in_specs=[
            pl.BlockSpec((n, in_ch), lambda i: (0, 0)),
            pl.BlockSpec((in_ch, out_ch), lambda i: (0, 0)),
            pl.BlockSpec((1, out_ch), lambda i: (0, 0)),
            pl.BlockSpec((bm, n), lambda i: (2 * i, 0)),
            pl.BlockSpec((bm, n), lambda i: (2 * i + 1, 0)),
            pl.BlockSpec((1, out_ch), lambda i: (0, 0)),
            pl.BlockSpec((1, 1), lambda i: (0, 0)),
        ],
        out_specs=pl.BlockSpec((2 * bm, out_ch), lambda i: (i, 0)),
        out_shape=jax.ShapeDtypeStruct((n, out_ch), jnp.float32),
        scratch_shapes=[pltpu.VMEM((n, out_ch), jnp.float32)],
    )(emb, W.T, b_fc.reshape(1, out_ch), meta, meta,
      bias.reshape(1, out_ch), prelu_a.reshape(1, 1))
    return out


# speedup vs baseline: 1.0303x; 1.0303x over previous
import jax
import jax.numpy as jnp
from jax.experimental import pallas as pl
from jax.experimental.pallas import tpu as pltpu


def _gconv_kernel(emb_ref, wt_ref, bfc_ref, meta_ref, bias_ref, a_ref,
                  out_ref, feat_ref):
    @pl.when(pl.program_id(0) == 0)
    def _():
        acc = jax.lax.dot_general(
            emb_ref[...].astype(jnp.bfloat16),
            wt_ref[...].astype(jnp.bfloat16),
            (((1,), (1,)), ((), ())),
            preferred_element_type=jnp.float32,
        )
        feat_ref[...] = acc + bfc_ref[...]

    acc = jax.lax.dot_general(
        meta_ref[...],
        feat_ref[...],
        (((1,), (0,)), ((), ())),
        precision=jax.lax.Precision.DEFAULT,
        preferred_element_type=jnp.float32,
    )
    r = acc + bias_ref[...]
    out_ref[...] = jnp.where(r >= 0, r, a_ref[0, 0] * r)


def kernel(emb, meta, W, b_fc, bias, prelu_a):
    n, in_ch = emb.shape
    out_ch = W.shape[0]

    bm = 400
    grid = (pl.cdiv(n, bm),)
    out = pl.pallas_call(
        _gconv_kernel,
        grid=grid,
        in_specs=[
            pl.BlockSpec((n, in_ch), lambda i: (0, 0)),
            pl.BlockSpec((in_ch, out_ch), lambda i: (0, 0)),
            pl.BlockSpec((1, out_ch), lambda i: (0, 0)),
            pl.BlockSpec((bm, n), lambda i: (i, 0)),
            pl.BlockSpec((1, out_ch), lambda i: (0, 0)),
            pl.BlockSpec((1, 1), lambda i: (0, 0)),
        ],
        out_specs=pl.BlockSpec((bm, out_ch), lambda i: (i, 0)),
        out_shape=jax.ShapeDtypeStruct((n, out_ch), jnp.float32),
        scratch_shapes=[pltpu.VMEM((n, out_ch), jnp.float32)],
    )(emb, W, b_fc.reshape(1, out_ch), meta, bias.reshape(1, out_ch),
      prelu_a.reshape(1, 1))
    return out


# speedup vs baseline: 1.0329x; 1.0025x over previous
import jax
import jax.numpy as jnp
from jax.experimental import pallas as pl
from jax.experimental.pallas import tpu as pltpu


def _gconv_kernel(emb_ref, w_ref, bfc_ref, meta_ref, bias_ref, a_ref,
                  out_ref, feat_ref):
    @pl.when(pl.program_id(0) == 0)
    def _():
        acc = jax.lax.dot_general(
            emb_ref[...].astype(jnp.bfloat16),
            w_ref[...].astype(jnp.bfloat16),
            (((1,), (1,)), ((), ())),
            preferred_element_type=jnp.float32,
        )
        feat_ref[...] = acc + bfc_ref[...]

    acc = jax.lax.dot_general(
        meta_ref[...],
        feat_ref[...],
        (((1,), (0,)), ((), ())),
        precision=jax.lax.Precision.DEFAULT,
        preferred_element_type=jnp.float32,
    )
    r = acc + bias_ref[...]
    out_ref[...] = jnp.where(r >= 0, r, a_ref[0, 0] * r)


def kernel(emb, meta, W, b_fc, bias, prelu_a):
    n, in_ch = emb.shape
    out_ch = W.shape[0]

    bm = 400
    grid = (pl.cdiv(n, bm),)
    out = pl.pallas_call(
        _gconv_kernel,
        grid=grid,
        in_specs=[
            pl.BlockSpec((n, in_ch), lambda i: (0, 0)),
            pl.BlockSpec((out_ch, in_ch), lambda i: (0, 0)),
            pl.BlockSpec((1, out_ch), lambda i: (0, 0)),
            pl.BlockSpec((bm, n), lambda i: (i, 0)),
            pl.BlockSpec((1, out_ch), lambda i: (0, 0)),
            pl.BlockSpec((1, 1), lambda i: (0, 0)),
        ],
        out_specs=pl.BlockSpec((bm, out_ch), lambda i: (i, 0)),
        out_shape=jax.ShapeDtypeStruct((n, out_ch), jnp.float32),
        scratch_shapes=[pltpu.VMEM((n, out_ch), jnp.float32)],
    )(emb, W, b_fc.reshape(1, out_ch), meta, bias.reshape(1, out_ch),
      prelu_a.reshape(1, 1))
    return out
